# baseline (speedup 1.0000x reference)
import functools

import jax
import jax.numpy as jnp
from jax import lax
from jax.experimental import pallas as pl
from jax.experimental.pallas import tpu as pltpu
from jax.experimental.pallas import tpu_sc as plsc

_D = 64
_B = 16384 * 20
_NC, _NS = 2, 16
_NW = _NC * _NS
_CHUNK = 128
_PER_W = _B // _NW
_NCHUNK = _PER_W // _CHUNK

_mesh = plsc.VectorSubcoreMesh(core_axis_name="c", subcore_axis_name="s")


@functools.partial(
    pl.kernel,
    out_type=jax.ShapeDtypeStruct((_B, _D), jnp.float32),
    mesh=_mesh,
    scratch_types=[
        pltpu.VMEM((_NCHUNK, _CHUNK), jnp.int32),
        pltpu.VMEM((_CHUNK, _D), jnp.float32),
        pltpu.SemaphoreType.DMA,
    ],
    compiler_params=pltpu.CompilerParams(use_tc_tiling_on_sc=False),
)
def _emb_lookup(idx_hbm, table_hbm, out_hbm, idx_v, rows_v, sem):
    wid = lax.axis_index("s") * _NC + lax.axis_index("c")
    pltpu.sync_copy(idx_hbm.at[pl.ds(wid * _NCHUNK, _NCHUNK)], idx_v)

    def body(j, carry):
        pltpu.async_copy(table_hbm.at[idx_v.at[j]], rows_v, sem).wait()
        row0 = wid * _PER_W + j * _CHUNK
        pltpu.sync_copy(rows_v, out_hbm.at[pl.ds(row0, _CHUNK)])
        return carry

    lax.fori_loop(0, _NCHUNK, body, 0)


def kernel(x, vocab):
    idx = x.reshape(_B // _CHUNK, _CHUNK)
    out = _emb_lookup(idx, vocab)
    return out.reshape(x.shape + (_D,))


# speedup vs baseline: 1.0636x; 1.0636x over previous
import functools

import jax
import jax.numpy as jnp
from jax import lax
from jax.experimental import pallas as pl
from jax.experimental.pallas import tpu as pltpu
from jax.experimental.pallas import tpu_sc as plsc

_D = 64
_B = 16384 * 20
_NC, _NS = 2, 16
_NW = _NC * _NS
_CHUNK = 128
_PER_W = _B // _NW
_NCHUNK = _PER_W // _CHUNK

_mesh = plsc.VectorSubcoreMesh(core_axis_name="c", subcore_axis_name="s")


_K = 4
_GROUP = _K * _CHUNK
_NGROUP = _PER_W // _GROUP
_NB = 2


@functools.partial(
    pl.kernel,
    out_type=jax.ShapeDtypeStruct((_B, _D), jnp.float32),
    mesh=_mesh,
    scratch_types=[
        pltpu.VMEM((_NCHUNK, _CHUNK), jnp.int32),
        pltpu.VMEM((_NB, _GROUP, _D), jnp.float32),
        pltpu.SemaphoreType.DMA,
        pltpu.SemaphoreType.DMA,
    ],
    compiler_params=pltpu.CompilerParams(use_tc_tiling_on_sc=False),
)
def _emb_lookup(idx_hbm, table_hbm, out_hbm, idx_v, rows_v, sem0, sem1):
    wid = lax.axis_index("s") * _NC + lax.axis_index("c")
    sems = (sem0, sem1)
    pltpu.sync_copy(idx_hbm.at[pl.ds(wid * _NCHUNK, _NCHUNK)], idx_v)

    def issue_group(g, b):
        for k in range(_K):
            pltpu.async_copy(
                table_hbm.at[idx_v.at[g * _K + k]],
                rows_v.at[b].at[pl.ds(k * _CHUNK, _CHUNK)],
                sems[b],
            )

    for b in range(_NB):
        issue_group(b, b)

    def body(gg, carry):
        for b in range(_NB):
            g = gg * _NB + b
            pltpu.make_async_copy(
                table_hbm.at[pl.ds(0, _GROUP)], rows_v.at[b], sems[b]
            ).wait()
            row0 = wid * _PER_W + g * _GROUP
            pltpu.sync_copy(rows_v.at[b], out_hbm.at[pl.ds(row0, _GROUP)])
            gn = g + _NB

            @pl.when(gn < _NGROUP)
            def _():
                issue_group(gn, b)

        return carry

    lax.fori_loop(0, _NGROUP // _NB, body, 0)


def kernel(x, vocab):
    idx = x.reshape(_B // _CHUNK, _CHUNK)
    out = _emb_lookup(idx, vocab)
    return out.reshape(x.shape + (_D,))
